# baseline (speedup 1.0000x reference)
import jax
import jax.numpy as jnp
from jax.experimental import pallas as pl
from jax.experimental.pallas import tpu as pltpu

N = 4096
F = 128
BI = 1024
BI_CAST = 512


def _hop0_body(gi, bi):
    def body(A, x0b, x1T, x1f, x0f, W1, b1, W2, b2,
             a16out, outl, outr, nl, nr, nlbfT, nrbf, yrT):
        i = pl.program_id(0)
        a = A[...].astype(jnp.bfloat16)
        a16out[...] = a

        W1v = W1[...].astype(jnp.bfloat16)
        W2v = W2[...].astype(jnp.bfloat16)
        bias = b1[...] + b2[...]

        ylv = jax.lax.dot_general(
            a, x0b[...], (((1,), (0,)), ((), ())),
            preferred_element_type=jnp.float32,
        )
        lm = ylv * x1f[...]
        outl[...] = (
            jnp.dot(ylv.astype(jnp.bfloat16), W1v, preferred_element_type=jnp.float32)
            + jnp.dot(lm.astype(jnp.bfloat16), W2v, preferred_element_type=jnp.float32)
            + bias
        )
        nx = ylv + lm
        nl[...] = nx
        nlbfT[...] = jnp.transpose(nx.astype(jnp.bfloat16))

        pT = jax.lax.dot_general(
            x1T[...], a, (((1,), (0,)), ((), ())),
            preferred_element_type=jnp.float32,
        )

        @pl.when(i == 0)
        def _():
            yrT[...] = pT

        @pl.when(i != 0)
        def _():
            yrT[...] += pT

        @pl.when(i == gi - 1)
        def _():
            yrv = jnp.transpose(yrT[...])
            rm = yrv * x0f[...]
            outr[...] = (
                jnp.dot(yrv.astype(jnp.bfloat16), W1v, preferred_element_type=jnp.float32)
                + jnp.dot(rm.astype(jnp.bfloat16), W2v, preferred_element_type=jnp.float32)
                + bias
            )
            nxr = yrv + rm
            nr[...] = nxr
            nrbf[...] = nxr.astype(jnp.bfloat16)

    return body


def _hop0(A, x0b, x1T, x1f, x0f, W1, b1, W2, b2):
    bi = BI_CAST
    gi = N // bi
    full = pl.BlockSpec((N, F), lambda i: (0, 0))
    strip = pl.BlockSpec((bi, F), lambda i: (i, 0))
    stripT = pl.BlockSpec((F, bi), lambda i: (0, i))
    a_strip = pl.BlockSpec((bi, N), lambda i: (i, 0))
    wspec = pl.BlockSpec((F, F), lambda i: (0, 0))
    bspec = pl.BlockSpec((1, F), lambda i: (0, 0))
    return pl.pallas_call(
        _hop0_body(gi, bi),
        grid=(gi,),
        in_specs=[a_strip, full, stripT, strip, full,
                  wspec, bspec, wspec, bspec],
        out_specs=(a_strip, strip, full, strip, full, stripT, full),
        out_shape=(
            jax.ShapeDtypeStruct((N, N), jnp.bfloat16),
            jax.ShapeDtypeStruct((N, F), jnp.float32),
            jax.ShapeDtypeStruct((N, F), jnp.float32),
            jax.ShapeDtypeStruct((N, F), jnp.float32),
            jax.ShapeDtypeStruct((N, F), jnp.float32),
            jax.ShapeDtypeStruct((F, N), jnp.bfloat16),
            jax.ShapeDtypeStruct((N, F), jnp.bfloat16),
        ),
        scratch_shapes=[pltpu.VMEM((F, N), jnp.float32)],
    )(A, x0b, x1T, x1f, x0f, W1, b1, W2, b2)


def _mega_body(gi, bi):
    def body(A16, ol0, or0, nl0, nr0, nlbfT0, nrbf0,
             W1a, b1a, W2a, b2a, W1b, b1b, W2b, b2b,
             y2out,
             yrT, y1, z1, ol2, nl2, y1bfT, z1bf, nr2bf):
        p = pl.program_id(0)
        i = pl.program_id(1)
        a = A16[...]
        sl_i = pl.ds(i * bi, bi)

        def rdot_accum(x1Tv):
            pT = jax.lax.dot_general(
                x1Tv, a, (((1,), (0,)), ((), ())),
                preferred_element_type=jnp.float32,
            )

            @pl.when(i == 0)
            def _():
                yrT[...] = pT

            @pl.when(i != 0)
            def _():
                yrT[...] += pT

        @pl.when(p == 0)
        def _():
            W1v = W1a[...].astype(jnp.bfloat16)
            W2v = W2a[...].astype(jnp.bfloat16)
            bias = b1a[...] + b2a[...]
            ylv = jax.lax.dot_general(
                a, nrbf0[...], (((1,), (0,)), ((), ())),
                preferred_element_type=jnp.float32,
            )
            lm = ylv * nl0[sl_i, :]
            olv = (
                jnp.dot(ylv.astype(jnp.bfloat16), W1v, preferred_element_type=jnp.float32)
                + jnp.dot(lm.astype(jnp.bfloat16), W2v, preferred_element_type=jnp.float32)
                + bias + ol0[sl_i, :]
            )
            y1v = jnp.maximum(olv, 0.0)
            y1[sl_i, :] = y1v
            y1bfT[:, sl_i] = jnp.transpose(y1v.astype(jnp.bfloat16))
            rdot_accum(nlbfT0[:, sl_i])

            @pl.when(i == gi - 1)
            def _():
                yrv = jnp.transpose(yrT[...])
                rm = yrv * nr0[...]
                orv = (
                    jnp.dot(yrv.astype(jnp.bfloat16), W1v, preferred_element_type=jnp.float32)
                    + jnp.dot(rm.astype(jnp.bfloat16), W2v, preferred_element_type=jnp.float32)
                    + bias + or0[...]
                )
                z1v = jnp.maximum(orv, 0.0)
                z1[...] = z1v
                z1bf[...] = z1v.astype(jnp.bfloat16)

        @pl.when(p == 1)
        def _():
            W1v = W1b[...].astype(jnp.bfloat16)
            W2v = W2b[...].astype(jnp.bfloat16)
            bias = b1b[...] + b2b[...]
            ylv = jax.lax.dot_general(
                a, z1bf[...], (((1,), (0,)), ((), ())),
                preferred_element_type=jnp.float32,
            )
            lm = ylv * y1[sl_i, :]
            ol2[sl_i, :] = (
                jnp.dot(ylv.astype(jnp.bfloat16), W1v, preferred_element_type=jnp.float32)
                + jnp.dot(lm.astype(jnp.bfloat16), W2v, preferred_element_type=jnp.float32)
                + bias
            )
            nl2[sl_i, :] = ylv + lm
            rdot_accum(y1bfT[:, sl_i])

            @pl.when(i == gi - 1)
            def _():
                yrv = jnp.transpose(yrT[...])
                nr2 = yrv + yrv * z1[...]
                nr2bf[...] = nr2.astype(jnp.bfloat16)

        @pl.when(p == 2)
        def _():
            W1v = W1b[...].astype(jnp.bfloat16)
            W2v = W2b[...].astype(jnp.bfloat16)
            bias = b1b[...] + b2b[...]
            ylv = jax.lax.dot_general(
                a, nr2bf[...], (((1,), (0,)), ((), ())),
                preferred_element_type=jnp.float32,
            )
            lm = ylv * nl2[sl_i, :]
            y2v = (
                jnp.dot(ylv.astype(jnp.bfloat16), W1v, preferred_element_type=jnp.float32)
                + jnp.dot(lm.astype(jnp.bfloat16), W2v, preferred_element_type=jnp.float32)
                + bias + ol2[sl_i, :]
            )
            y2out[...] = jnp.maximum(y2v, 0.0)

    return body


def _mega(A16, ol0, or0, nl0, nr0, nlbfT0, nrbf0,
          W1a, b1a, W2a, b2a, W1b, b1b, W2b, b2b):
    gi = N // BI
    a_spec = pl.BlockSpec((BI, N), lambda p, i: (i, 0))
    full = pl.BlockSpec((N, F), lambda p, i: (0, 0))
    fullT = pl.BlockSpec((F, N), lambda p, i: (0, 0))
    wspec = pl.BlockSpec((F, F), lambda p, i: (0, 0))
    bspec = pl.BlockSpec((1, F), lambda p, i: (0, 0))
    in_specs = ([a_spec] + [full] * 4 + [fullT, full]
                + [wspec, bspec, wspec, bspec] * 2)
    scratch = [
        pltpu.VMEM((F, N), jnp.float32),
        pltpu.VMEM((N, F), jnp.float32),
        pltpu.VMEM((N, F), jnp.float32),
        pltpu.VMEM((N, F), jnp.float32),
        pltpu.VMEM((N, F), jnp.float32),
        pltpu.VMEM((F, N), jnp.bfloat16),
        pltpu.VMEM((N, F), jnp.bfloat16),
        pltpu.VMEM((N, F), jnp.bfloat16),
    ]
    return pl.pallas_call(
        _mega_body(gi, BI),
        grid=(3, gi),
        in_specs=in_specs,
        out_specs=pl.BlockSpec((BI, F), lambda p, i: (i, 0)),
        out_shape=jax.ShapeDtypeStruct((N, F), jnp.float32),
        scratch_shapes=scratch,
    )(A16, ol0, or0, nl0, nr0, nlbfT0, nrbf0,
      W1a, b1a, W2a, b2a, W1b, b1b, W2b, b2b)


def kernel(l_feat, r_feat, network, W1a, b1a, W2a, b2a, W1b, b1b, W2b, b2b):
    rbf = r_feat.astype(jnp.bfloat16)
    lTbf = l_feat.astype(jnp.bfloat16).T
    b1a = b1a.reshape(1, F)
    b2a = b2a.reshape(1, F)
    b1b = b1b.reshape(1, F)
    b2b = b2b.reshape(1, F)

    A16, ol, orv, nl, nr, nlbfT, nrbf = _hop0(
        network, rbf, lTbf, l_feat, r_feat, W1a, b1a, W2a, b2a,
    )
    return _mega(
        A16, ol, orv, nl, nr, nlbfT, nrbf,
        W1a, b1a, W2a, b2a, W1b, b1b, W2b, b2b,
    )


# speedup vs baseline: 1.1373x; 1.1373x over previous
import jax
import jax.numpy as jnp
from jax.experimental import pallas as pl
from jax.experimental.pallas import tpu as pltpu

N = 4096
F = 128
BI = 1024
BI_CAST = 512


def _hop0_body(gi, bi):
    def body(A, x0b, x1T, x1f, x0f, W1, b1, W2, b2,
             a16out, outl, outr, nl, nr, nlbfT, nrbf, yrT):
        i = pl.program_id(0)
        a = A[...].astype(jnp.bfloat16)
        a16out[...] = a

        W1v = W1[...].astype(jnp.bfloat16)
        W2v = W2[...].astype(jnp.bfloat16)
        bias = b1[...] + b2[...]

        pT = jax.lax.dot_general(
            x1T[...], a, (((1,), (0,)), ((), ())),
            preferred_element_type=jnp.float32,
        )

        @pl.when(i == 0)
        def _():
            yrT[...] = pT

        @pl.when(i != 0)
        def _():
            yrT[...] += pT

        ylv = jax.lax.dot_general(
            a, x0b[...], (((1,), (0,)), ((), ())),
            preferred_element_type=jnp.float32,
        )
        lm = ylv * x1f[...]
        outl[...] = (
            jnp.dot(ylv.astype(jnp.bfloat16), W1v, preferred_element_type=jnp.float32)
            + jnp.dot(lm.astype(jnp.bfloat16), W2v, preferred_element_type=jnp.float32)
            + bias
        )
        nx = ylv + lm
        nl[...] = nx
        nlbfT[...] = jnp.transpose(nx.astype(jnp.bfloat16))

        @pl.when(i == gi - 1)
        def _():
            yrv = jnp.transpose(yrT[...])
            rm = yrv * x0f[...]
            outr[...] = (
                jnp.dot(yrv.astype(jnp.bfloat16), W1v, preferred_element_type=jnp.float32)
                + jnp.dot(rm.astype(jnp.bfloat16), W2v, preferred_element_type=jnp.float32)
                + bias
            )
            nxr = yrv + rm
            nr[...] = nxr
            nrbf[...] = nxr.astype(jnp.bfloat16)

    return body


def _hop0(A, x0b, x1T, x1f, x0f, W1, b1, W2, b2):
    bi = BI_CAST
    gi = N // bi
    full = pl.BlockSpec((N, F), lambda i: (0, 0))
    strip = pl.BlockSpec((bi, F), lambda i: (i, 0))
    stripT = pl.BlockSpec((F, bi), lambda i: (0, i))
    a_strip = pl.BlockSpec((bi, N), lambda i: (i, 0))
    wspec = pl.BlockSpec((F, F), lambda i: (0, 0))
    bspec = pl.BlockSpec((1, F), lambda i: (0, 0))
    return pl.pallas_call(
        _hop0_body(gi, bi),
        grid=(gi,),
        in_specs=[a_strip, full, stripT, strip, full,
                  wspec, bspec, wspec, bspec],
        out_specs=(a_strip, strip, full, strip, full, stripT, full),
        out_shape=(
            jax.ShapeDtypeStruct((N, N), jnp.bfloat16),
            jax.ShapeDtypeStruct((N, F), jnp.float32),
            jax.ShapeDtypeStruct((N, F), jnp.float32),
            jax.ShapeDtypeStruct((N, F), jnp.float32),
            jax.ShapeDtypeStruct((N, F), jnp.float32),
            jax.ShapeDtypeStruct((F, N), jnp.bfloat16),
            jax.ShapeDtypeStruct((N, F), jnp.bfloat16),
        ),
        scratch_shapes=[pltpu.VMEM((F, N), jnp.float32)],
    )(A, x0b, x1T, x1f, x0f, W1, b1, W2, b2)


def _mega_body(gi, bi):
    def body(A16, ol0, or0, nl0, nr0, nlbfT0, nrbf0,
             W1a, b1a, W2a, b2a, W1b, b1b, W2b, b2b,
             y2out,
             yrT, y1, z1, ol2, nl2, y1bfT, z1bf, nr2bf):
        p = pl.program_id(0)
        i = pl.program_id(1)
        a = A16[...]
        sl_i = pl.ds(i * bi, bi)

        def rdot_accum(x1Tv):
            pT = jax.lax.dot_general(
                x1Tv, a, (((1,), (0,)), ((), ())),
                preferred_element_type=jnp.float32,
            )

            @pl.when(i == 0)
            def _():
                yrT[...] = pT

            @pl.when(i != 0)
            def _():
                yrT[...] += pT

        @pl.when(p == 0)
        def _():
            W1v = W1a[...].astype(jnp.bfloat16)
            W2v = W2a[...].astype(jnp.bfloat16)
            bias = b1a[...] + b2a[...]
            rdot_accum(nlbfT0[:, sl_i])
            ylv = jax.lax.dot_general(
                a, nrbf0[...], (((1,), (0,)), ((), ())),
                preferred_element_type=jnp.float32,
            )
            lm = ylv * nl0[sl_i, :]
            olv = (
                jnp.dot(ylv.astype(jnp.bfloat16), W1v, preferred_element_type=jnp.float32)
                + jnp.dot(lm.astype(jnp.bfloat16), W2v, preferred_element_type=jnp.float32)
                + bias + ol0[sl_i, :]
            )
            y1v = jnp.maximum(olv, 0.0)
            y1[sl_i, :] = y1v
            y1bfT[:, sl_i] = jnp.transpose(y1v.astype(jnp.bfloat16))

            @pl.when(i == gi - 1)
            def _():
                yrv = jnp.transpose(yrT[...])
                rm = yrv * nr0[...]
                orv = (
                    jnp.dot(yrv.astype(jnp.bfloat16), W1v, preferred_element_type=jnp.float32)
                    + jnp.dot(rm.astype(jnp.bfloat16), W2v, preferred_element_type=jnp.float32)
                    + bias + or0[...]
                )
                z1v = jnp.maximum(orv, 0.0)
                z1[...] = z1v
                z1bf[...] = z1v.astype(jnp.bfloat16)

        @pl.when(p == 1)
        def _():
            W1v = W1b[...].astype(jnp.bfloat16)
            W2v = W2b[...].astype(jnp.bfloat16)
            bias = b1b[...] + b2b[...]
            rdot_accum(y1bfT[:, sl_i])
            ylv = jax.lax.dot_general(
                a, z1bf[...], (((1,), (0,)), ((), ())),
                preferred_element_type=jnp.float32,
            )
            lm = ylv * y1[sl_i, :]
            ol2[sl_i, :] = (
                jnp.dot(ylv.astype(jnp.bfloat16), W1v, preferred_element_type=jnp.float32)
                + jnp.dot(lm.astype(jnp.bfloat16), W2v, preferred_element_type=jnp.float32)
                + bias
            )
            nl2[sl_i, :] = ylv + lm

            @pl.when(i == gi - 1)
            def _():
                yrv = jnp.transpose(yrT[...])
                nr2 = yrv + yrv * z1[...]
                nr2bf[...] = nr2.astype(jnp.bfloat16)

        @pl.when(p == 2)
        def _():
            W1v = W1b[...].astype(jnp.bfloat16)
            W2v = W2b[...].astype(jnp.bfloat16)
            bias = b1b[...] + b2b[...]
            ylv = jax.lax.dot_general(
                a, nr2bf[...], (((1,), (0,)), ((), ())),
                preferred_element_type=jnp.float32,
            )
            lm = ylv * nl2[sl_i, :]
            y2v = (
                jnp.dot(ylv.astype(jnp.bfloat16), W1v, preferred_element_type=jnp.float32)
                + jnp.dot(lm.astype(jnp.bfloat16), W2v, preferred_element_type=jnp.float32)
                + bias + ol2[sl_i, :]
            )
            y2out[...] = jnp.maximum(y2v, 0.0)

    return body


def _mega(A16, ol0, or0, nl0, nr0, nlbfT0, nrbf0,
          W1a, b1a, W2a, b2a, W1b, b1b, W2b, b2b):
    gi = N // BI
    a_spec = pl.BlockSpec((BI, N), lambda p, i: (i, 0))
    full = pl.BlockSpec((N, F), lambda p, i: (0, 0))
    fullT = pl.BlockSpec((F, N), lambda p, i: (0, 0))
    wspec = pl.BlockSpec((F, F), lambda p, i: (0, 0))
    bspec = pl.BlockSpec((1, F), lambda p, i: (0, 0))
    in_specs = ([a_spec] + [full] * 4 + [fullT, full]
                + [wspec, bspec, wspec, bspec] * 2)
    scratch = [
        pltpu.VMEM((F, N), jnp.float32),
        pltpu.VMEM((N, F), jnp.float32),
        pltpu.VMEM((N, F), jnp.float32),
        pltpu.VMEM((N, F), jnp.float32),
        pltpu.VMEM((N, F), jnp.float32),
        pltpu.VMEM((F, N), jnp.bfloat16),
        pltpu.VMEM((N, F), jnp.bfloat16),
        pltpu.VMEM((N, F), jnp.bfloat16),
    ]
    return pl.pallas_call(
        _mega_body(gi, BI),
        grid=(3, gi),
        in_specs=in_specs,
        out_specs=pl.BlockSpec((BI, F), lambda p, i: (i, 0)),
        out_shape=jax.ShapeDtypeStruct((N, F), jnp.float32),
        scratch_shapes=scratch,
    )(A16, ol0, or0, nl0, nr0, nlbfT0, nrbf0,
      W1a, b1a, W2a, b2a, W1b, b1b, W2b, b2b)


def kernel(l_feat, r_feat, network, W1a, b1a, W2a, b2a, W1b, b1b, W2b, b2b):
    rbf = r_feat.astype(jnp.bfloat16)
    lTbf = l_feat.astype(jnp.bfloat16).T
    b1a = b1a.reshape(1, F)
    b2a = b2a.reshape(1, F)
    b1b = b1b.reshape(1, F)
    b2b = b2b.reshape(1, F)

    A16, ol, orv, nl, nr, nlbfT, nrbf = _hop0(
        network, rbf, lTbf, l_feat, r_feat, W1a, b1a, W2a, b2a,
    )
    return _mega(
        A16, ol, orv, nl, nr, nlbfT, nrbf,
        W1a, b1a, W2a, b2a, W1b, b1b, W2b, b2b,
    )
